# Initial kernel scaffold; baseline (speedup 1.0000x reference)
import jax
import jax.numpy as jnp
from jax.experimental import pallas as pl


def kernel(x, edge_index, W_rel0, b_rel0, W_root0, W_rel1, b_rel1, W_root1, W_rel2, b_rel2, W_root2, W_lin, b_lin):
    raise NotImplementedError("write your pallas kernel here")


# speedup vs baseline: 3.8730x; 3.8730x over previous
import functools

import jax
import jax.numpy as jnp
from jax import lax
from jax.experimental import pallas as pl
from jax.experimental.pallas import tpu as pltpu
from jax.experimental.pallas import tpu_sc as plsc

N = 10000
NPAD = 10112
E = 160000
NSC = 2
NSUB = 16
EPT = E // NSUB
CHUNK = 128
NCHUNK = 79
EPT_PAD = NCHUNK * CHUNK
ROWS_PER_SUB = NPAD // NSUB
H = 192
HW = 128


def _sc_aggregate(hr_a, hr_b, srcp, dstp, zeros):
    mesh = plsc.VectorSubcoreMesh(core_axis_name="c", subcore_axis_name="s")

    @functools.partial(
        pl.kernel,
        mesh=mesh,
        out_type=jax.ShapeDtypeStruct((NSC, NPAD, HW), jnp.float32),
        scratch_types=[
            pltpu.VMEM((NCHUNK, CHUNK), jnp.int32),
            pltpu.VMEM((NCHUNK, CHUNK), jnp.int32),
            pltpu.VMEM((CHUNK, HW), jnp.float32),
            pltpu.VMEM_SHARED((NPAD, HW), jnp.float32),
            pltpu.SemaphoreType.DMA,
        ],
    )
    def agg_kernel(hra_hbm, hrb_hbm, src_hbm, dst_hbm, zeros_hbm, out_hbm,
                   src_v, dst_v, rows_v, acc, sem):
        c = lax.axis_index("c")
        s = lax.axis_index("s")
        pltpu.sync_copy(zeros_hbm, acc.at[pl.ds(s * ROWS_PER_SUB, ROWS_PER_SUB)])
        pltpu.sync_copy(src_hbm.at[s], src_v)
        pltpu.sync_copy(dst_hbm.at[s], dst_v)
        plsc.subcore_barrier()

        def body(j, carry):
            @pl.when(c == 0)
            def _():
                pltpu.async_copy(hra_hbm.at[src_v.at[j]], rows_v, sem).wait()

            @pl.when(c == 1)
            def _():
                pltpu.async_copy(hrb_hbm.at[src_v.at[j]], rows_v, sem).wait()

            pltpu.sync_copy(rows_v, acc.at[dst_v.at[j]], add=True)
            return carry

        lax.fori_loop(0, NCHUNK, body, 0)
        plsc.subcore_barrier()
        pltpu.sync_copy(acc.at[pl.ds(s * ROWS_PER_SUB, ROWS_PER_SUB)],
                        out_hbm.at[c, pl.ds(s * ROWS_PER_SUB, ROWS_PER_SUB)])

    return agg_kernel(hr_a, hr_b, srcp, dstp, zeros)


def _tc_first(x, Wr0p):
    BLK = 1000
    d = x.shape[1]

    def mm(x_ref, w_ref, oa_ref, ob_ref):
        r = lax.dot_general(
            x_ref[...], w_ref[...], (((1,), (1,)), ((), ())),
            preferred_element_type=jnp.float32)
        oa_ref[...] = r[:, :HW]
        ob_ref[...] = r[:, HW:]

    return pl.pallas_call(
        mm,
        grid=(N // BLK,),
        in_specs=[pl.BlockSpec((BLK, d), lambda i: (i, 0)),
                  pl.BlockSpec(Wr0p.shape, lambda i: (0, 0))],
        out_specs=[pl.BlockSpec((BLK, HW), lambda i: (i, 0)),
                   pl.BlockSpec((BLK, HW), lambda i: (i, 0))],
        out_shape=[jax.ShapeDtypeStruct((N, HW), jnp.float32),
                   jax.ShapeDtypeStruct((N, HW), jnp.float32)],
    )(x, Wr0p)


def _tc_layer(aggs, h, Wroot, br, Wnextp):
    BLK = 1000
    d = h.shape[1]

    def k(agg_ref, h_ref, wroot_ref, br_ref, wnext_ref,
          hnew_ref, hra_ref, hrb_ref):
        agg = jnp.concatenate([agg_ref[0], agg_ref[1][:, :H - HW]], axis=1)
        root = lax.dot_general(
            h_ref[...], wroot_ref[...], (((1,), (1,)), ((), ())),
            preferred_element_type=jnp.float32)
        hnew = jnp.maximum(agg + root + br_ref[...], 0.0)
        hnew_ref[...] = hnew
        r = lax.dot_general(
            hnew, wnext_ref[...], (((1,), (1,)), ((), ())),
            preferred_element_type=jnp.float32)
        hra_ref[...] = r[:, :HW]
        hrb_ref[...] = r[:, HW:]

    return pl.pallas_call(
        k,
        grid=(N // BLK,),
        in_specs=[pl.BlockSpec((NSC, BLK, HW), lambda i: (0, i, 0)),
                  pl.BlockSpec((BLK, d), lambda i: (i, 0)),
                  pl.BlockSpec((H, d), lambda i: (0, 0)),
                  pl.BlockSpec((1, H), lambda i: (0, 0)),
                  pl.BlockSpec((2 * HW, H), lambda i: (0, 0))],
        out_specs=[pl.BlockSpec((BLK, H), lambda i: (i, 0)),
                   pl.BlockSpec((BLK, HW), lambda i: (i, 0)),
                   pl.BlockSpec((BLK, HW), lambda i: (i, 0))],
        out_shape=[jax.ShapeDtypeStruct((N, H), jnp.float32),
                   jax.ShapeDtypeStruct((N, HW), jnp.float32),
                   jax.ShapeDtypeStruct((N, HW), jnp.float32)],
    )(aggs, h, Wroot, br, Wnextp)


def _tc_final(aggs, h, Wroot, br, Wlin, blin):
    BLK = 1000
    d = h.shape[1]
    DO = Wlin.shape[0]

    def k(agg_ref, h_ref, wroot_ref, br_ref, wlin_ref, blin_ref, o_ref):
        agg = jnp.concatenate([agg_ref[0], agg_ref[1][:, :H - HW]], axis=1)
        root = lax.dot_general(
            h_ref[...], wroot_ref[...], (((1,), (1,)), ((), ())),
            preferred_element_type=jnp.float32)
        hnew = jnp.maximum(agg + root + br_ref[...], 0.0)
        o_ref[...] = lax.dot_general(
            hnew, wlin_ref[...], (((1,), (1,)), ((), ())),
            preferred_element_type=jnp.float32) + blin_ref[...]

    return pl.pallas_call(
        k,
        grid=(N // BLK,),
        in_specs=[pl.BlockSpec((NSC, BLK, HW), lambda i: (0, i, 0)),
                  pl.BlockSpec((BLK, d), lambda i: (i, 0)),
                  pl.BlockSpec((H, d), lambda i: (0, 0)),
                  pl.BlockSpec((1, H), lambda i: (0, 0)),
                  pl.BlockSpec((DO, H), lambda i: (0, 0)),
                  pl.BlockSpec((1, DO), lambda i: (0, 0))],
        out_specs=pl.BlockSpec((BLK, DO), lambda i: (i, 0)),
        out_shape=jax.ShapeDtypeStruct((N, DO), jnp.float32),
    )(aggs, h, Wroot, br, Wlin, blin)


def _pad_w(Wr):
    return jnp.pad(Wr, ((0, 2 * HW - H), (0, 0)))


def kernel(x, edge_index, W_rel0, b_rel0, W_root0, W_rel1, b_rel1, W_root1,
           W_rel2, b_rel2, W_root2, W_lin, b_lin):
    src = edge_index[0]
    dst = edge_index[1]
    pad = EPT_PAD * NSUB - E
    srcp = jnp.pad(src, (0, pad), constant_values=0
                   ).reshape(NSUB, NCHUNK, CHUNK)
    dstp = jnp.pad(dst, (0, pad), constant_values=N
                   ).reshape(NSUB, NCHUNK, CHUNK)
    zeros = jnp.zeros((ROWS_PER_SUB, HW), jnp.float32)

    hra0, hrb0 = _tc_first(x, _pad_w(W_rel0))
    agg0 = _sc_aggregate(hra0, hrb0, srcp, dstp, zeros)
    h1, hra1, hrb1 = _tc_layer(agg0, x, W_root0, b_rel0.reshape(1, -1),
                               _pad_w(W_rel1))
    agg1 = _sc_aggregate(hra1, hrb1, srcp, dstp, zeros)
    h2, hra2, hrb2 = _tc_layer(agg1, h1, W_root1, b_rel1.reshape(1, -1),
                               _pad_w(W_rel2))
    agg2 = _sc_aggregate(hra2, hrb2, srcp, dstp, zeros)
    return _tc_final(agg2, h2, W_root2, b_rel2.reshape(1, -1),
                     W_lin, b_lin.reshape(1, -1))
